# baseline (speedup 1.0000x reference)
import functools

import jax
import jax.numpy as jnp
from jax import lax
from jax.experimental import pallas as pl
from jax.experimental.pallas import tpu as pltpu
from jax.experimental.pallas import tpu_sc as plsc

D = 128
V = 98
VP = 104
S = 32
L = 16
NC = 2
NS = 16
NW = NC * NS
B = 16384
TOK = B * S
R = TOK // NW // 128


def _mesh():
    return plsc.VectorSubcoreMesh(
        core_axis_name="c", subcore_axis_name="s", num_cores=NC, num_subcores=NS
    )


def _pos_encoding():
    positions = jnp.arange(S, dtype=jnp.float32)
    power_values = jnp.power(
        1000.0, 2.0 * jnp.arange(0, D, 2, dtype=jnp.float32) / D
    )
    angle = positions[:, None] / power_values[None, :]
    pe = jnp.zeros((S, D), dtype=jnp.float32)
    pe = pe.at[:, 0::2].set(jnp.sin(angle))
    pe = pe.at[:, 1::2].set(jnp.cos(angle))
    return pe


def _embed(x2, W, pe):
    @functools.partial(
        pl.kernel,
        out_type=jax.ShapeDtypeStruct((TOK, D), jnp.float32),
        mesh=_mesh(),
        scratch_types=[
            pltpu.VMEM_SHARED((S * VP, D), jnp.float32),
            pltpu.VMEM((VP, D), jnp.float32),
            pltpu.VMEM((S, D), jnp.float32),
            pltpu.VMEM((R, 128), jnp.int32),
            pltpu.VMEM((4, 128, D), jnp.float32),
            pltpu.SemaphoreType.DMA,
            pltpu.SemaphoreType.DMA,
        ],
    )
    def k(x_hbm, w_hbm, pe_hbm, out_hbm, tsh, wv, pev, xi, data, gsem, wsem):
        cid = lax.axis_index("c")
        sid = lax.axis_index("s")
        wid = sid * NC + cid

        pltpu.sync_copy(pe_hbm, pev)
        for half in range(2):
            s = sid + half * NS
            pltpu.sync_copy(w_hbm, wv.at[pl.ds(0, V)])

            @pl.loop(0, V)
            def _(r):
                for c in range(D // L):
                    sl = pl.ds(c * L, L)
                    wv[r, sl] = wv[r, sl] + pev[s, sl]

            pltpu.sync_copy(wv, tsh.at[pl.ds(s * VP, VP)])

        pltpu.sync_copy(x_hbm.at[pl.ds(wid * R, R)], xi)
        iota = lax.broadcasted_iota(jnp.int32, (L,), 0)
        e0 = iota * VP
        e1 = (iota + L) * VP

        @pl.loop(0, R)
        def _(p):
            for c in range(D // L):
                sl = pl.ds(c * L, L)
                xi[p, sl] = xi[p, sl] + (e0 if c % 2 == 0 else e1)

        plsc.subcore_barrier()

        outbase = wid * (R * 128)

        def g_start(k, b):
            pltpu.async_copy(tsh.at[xi.at[k]], data.at[b], gsem)

        def g_wait(k, b):
            pltpu.make_async_copy(tsh.at[xi.at[k]], data.at[b], gsem).wait()

        def w_start(k, b):
            pltpu.async_copy(
                data.at[b], out_hbm.at[pl.ds(outbase + k * 128, 128)], wsem
            )

        def w_wait(k, b):
            pltpu.make_async_copy(
                data.at[b], out_hbm.at[pl.ds(outbase + k * 128, 128)], wsem
            ).wait()

        g_start(0, 0)
        for k in (0, 1, 2):
            g_wait(k, k)
            w_start(k, k)
            g_start(k + 1, k + 1)

        @pl.loop(3, R - 1, step=4)
        def _(g):
            for b in range(4):
                k = g + b
                bb = (3 + b) % 4
                g_wait(k, bb)
                w_start(k, bb)
                w_wait(k - 3, b % 4)
                g_start(k + 1, b % 4)

        g_wait(R - 1, 3)
        w_start(R - 1, 3)
        w_wait(R - 4, 0)
        w_wait(R - 3, 1)
        w_wait(R - 2, 2)
        w_wait(R - 1, 3)

    return k(x2, W, pe)


def kernel(x, start_token, end_token, W):
    del start_token, end_token
    x2 = x.reshape(TOK // 128, 128)
    out = _embed(x2, W, _pos_encoding())
    return out.reshape(B, S, D)


# speedup vs baseline: 1.0630x; 1.0630x over previous
import functools

import jax
import jax.numpy as jnp
from jax import lax
from jax.experimental import pallas as pl
from jax.experimental.pallas import tpu as pltpu
from jax.experimental.pallas import tpu_sc as plsc

D = 128
V = 98
VP = 104
S = 32
L = 16
NC = 2
NS = 16
NW = NC * NS
B = 16384
TOK = B * S
R = TOK // NW // 128


def _mesh():
    return plsc.VectorSubcoreMesh(
        core_axis_name="c", subcore_axis_name="s", num_cores=NC, num_subcores=NS
    )


def _pos_encoding():
    positions = jnp.arange(S, dtype=jnp.float32)
    power_values = jnp.power(
        1000.0, 2.0 * jnp.arange(0, D, 2, dtype=jnp.float32) / D
    )
    angle = positions[:, None] / power_values[None, :]
    pe = jnp.zeros((S, D), dtype=jnp.float32)
    pe = pe.at[:, 0::2].set(jnp.sin(angle))
    pe = pe.at[:, 1::2].set(jnp.cos(angle))
    return pe


def _embed(x2, W, pe):
    @functools.partial(
        pl.kernel,
        out_type=jax.ShapeDtypeStruct((TOK, D), jnp.float32),
        mesh=_mesh(),
        scratch_types=[
            pltpu.VMEM_SHARED((S * VP, D), jnp.float32),
            pltpu.VMEM((VP, D), jnp.float32),
            pltpu.VMEM((S, D), jnp.float32),
            pltpu.VMEM((R, 128), jnp.int32),
            pltpu.VMEM((4, 128, D), jnp.float32),
            pltpu.SemaphoreType.DMA,
            pltpu.SemaphoreType.DMA,
        ],
    )
    def k(x_hbm, w_hbm, pe_hbm, out_hbm, tsh, wv, pev, xi, data, gsem, wsem):
        cid = lax.axis_index("c")
        sid = lax.axis_index("s")
        wid = sid * NC + cid

        pltpu.sync_copy(pe_hbm, pev)
        for half in range(2):
            s = sid + half * NS
            pltpu.sync_copy(w_hbm, wv.at[pl.ds(0, V)])

            @pl.loop(0, V, unroll=7)
            def _(r):
                for c in range(D // L):
                    sl = pl.ds(c * L, L)
                    wv[r, sl] = wv[r, sl] + pev[s, sl]

            pltpu.sync_copy(wv, tsh.at[pl.ds(s * VP, VP)])

        pltpu.sync_copy(x_hbm.at[pl.ds(wid * R, R)], xi)
        iota = lax.broadcasted_iota(jnp.int32, (L,), 0)
        e0 = iota * VP
        e1 = (iota + L) * VP

        @pl.loop(0, R, unroll=8)
        def _(p):
            for c in range(D // L):
                sl = pl.ds(c * L, L)
                xi[p, sl] = xi[p, sl] + (e0 if c % 2 == 0 else e1)

        plsc.subcore_barrier()

        outbase = wid * (R * 128)

        def g_start(k, b):
            pltpu.async_copy(tsh.at[xi.at[k]], data.at[b], gsem)

        def g_wait(k, b):
            pltpu.make_async_copy(tsh.at[xi.at[k]], data.at[b], gsem).wait()

        def w_start(k, b):
            pltpu.async_copy(
                data.at[b], out_hbm.at[pl.ds(outbase + k * 128, 128)], wsem
            )

        def w_wait(k, b):
            pltpu.make_async_copy(
                data.at[b], out_hbm.at[pl.ds(outbase + k * 128, 128)], wsem
            ).wait()

        g_start(0, 0)
        g_start(1, 1)
        for k in (0, 1):
            g_wait(k, k)
            w_start(k, k)
            g_start(k + 2, k + 2)

        @pl.loop(2, R - 2, step=4)
        def _(g):
            for b in range(4):
                k = g + b
                bb = (2 + b) % 4
                g_wait(k, bb)
                w_start(k, bb)
                w_wait(k - 2, b % 4)
                g_start(k + 2, b % 4)

        for k, b in ((R - 2, 2), (R - 1, 3)):
            g_wait(k, b)
            w_start(k, b)
            w_wait(k - 2, (b + 2) % 4)
        w_wait(R - 2, 2)
        w_wait(R - 1, 3)

    return k(x2, W, pe)


def kernel(x, start_token, end_token, W):
    del start_token, end_token
    x2 = x.reshape(TOK // 128, 128)
    out = _embed(x2, W, _pos_encoding())
    return out.reshape(B, S, D)


# speedup vs baseline: 1.0694x; 1.0060x over previous
import functools

import jax
import jax.numpy as jnp
from jax import lax
from jax.experimental import pallas as pl
from jax.experimental.pallas import tpu as pltpu
from jax.experimental.pallas import tpu_sc as plsc

D = 128
V = 98
VP = 104
S = 32
L = 16
NC = 2
NS = 16
NW = NC * NS
B = 16384
TOK = B * S
R = TOK // NW // 128


def _mesh():
    return plsc.VectorSubcoreMesh(
        core_axis_name="c", subcore_axis_name="s", num_cores=NC, num_subcores=NS
    )


def _pos_encoding():
    positions = jnp.arange(S, dtype=jnp.float32)
    power_values = jnp.power(
        1000.0, 2.0 * jnp.arange(0, D, 2, dtype=jnp.float32) / D
    )
    angle = positions[:, None] / power_values[None, :]
    pe = jnp.zeros((S, D), dtype=jnp.float32)
    pe = pe.at[:, 0::2].set(jnp.sin(angle))
    pe = pe.at[:, 1::2].set(jnp.cos(angle))
    return pe


def _embed(x2, W, pe):
    @functools.partial(
        pl.kernel,
        out_type=jax.ShapeDtypeStruct((TOK, D), jnp.float32),
        mesh=_mesh(),
        scratch_types=[
            pltpu.VMEM_SHARED((S * VP, D), jnp.float32),
            pltpu.VMEM((VP, D), jnp.float32),
            pltpu.VMEM((S, D), jnp.float32),
            pltpu.VMEM((R, 128), jnp.int32),
            pltpu.VMEM((4, 128, D), jnp.float32),
            pltpu.SemaphoreType.DMA,
            pltpu.SemaphoreType.DMA,
            pltpu.SemaphoreType.DMA,
            pltpu.SemaphoreType.DMA,
            pltpu.SemaphoreType.DMA,
        ],
    )
    def k(x_hbm, w_hbm, pe_hbm, out_hbm, tsh, tb, pev, xi, data,
          gsem, wsem, xsem, lsem, tsem):
        cid = lax.axis_index("c")
        sid = lax.axis_index("s")
        wid = sid * NC + cid

        cp_x = pltpu.async_copy(x_hbm.at[pl.ds(wid * R, R)], xi, xsem)
        cp_pe = pltpu.async_copy(pe_hbm, pev, lsem)
        cp_w = pltpu.async_copy(w_hbm, tb.at[pl.ds(0, V)], lsem)
        cp_pe.wait()
        cp_w.wait()
        s0 = sid
        s1 = sid + NS

        @pl.loop(0, V)
        def _(r):
            for c in range(D // L):
                sl = pl.ds(c * L, L)
                tb[r, sl] = tb[r, sl] + pev[s0, sl]

        pltpu.sync_copy(tb, tsh.at[pl.ds(s0 * VP, VP)])
        cp_w = pltpu.async_copy(w_hbm, tb.at[pl.ds(0, V)], lsem)
        cp_w.wait()

        @pl.loop(0, V)
        def _(r):
            for c in range(D // L):
                sl = pl.ds(c * L, L)
                tb[r, sl] = tb[r, sl] + pev[s1, sl]

        pltpu.async_copy(tb, tsh.at[pl.ds(s1 * VP, VP)], tsem)

        cp_x.wait()
        iota = lax.broadcasted_iota(jnp.int32, (L,), 0)
        e0 = iota * VP
        e1 = (iota + L) * VP

        @pl.loop(0, R)
        def _(p):
            for c in range(D // L):
                sl = pl.ds(c * L, L)
                xi[p, sl] = xi[p, sl] + (e0 if c % 2 == 0 else e1)

        pltpu.make_async_copy(tb, tsh.at[pl.ds(s1 * VP, VP)], tsem).wait()

        plsc.subcore_barrier()

        outbase = wid * (R * 128)

        def g_start(k, b):
            pltpu.async_copy(tsh.at[xi.at[k]], data.at[b], gsem)

        def g_wait(k, b):
            pltpu.make_async_copy(tsh.at[xi.at[k]], data.at[b], gsem).wait()

        def w_start(k, b):
            pltpu.async_copy(
                data.at[b], out_hbm.at[pl.ds(outbase + k * 128, 128)], wsem
            )

        def w_wait(k, b):
            pltpu.make_async_copy(
                data.at[b], out_hbm.at[pl.ds(outbase + k * 128, 128)], wsem
            ).wait()

        g_start(0, 0)
        g_start(1, 1)
        for k in (0, 1):
            g_wait(k, k)
            w_start(k, k)
            g_start(k + 2, k + 2)

        @pl.loop(2, R - 2, step=4)
        def _(g):
            for b in range(4):
                k = g + b
                bb = (2 + b) % 4
                g_wait(k, bb)
                w_start(k, bb)
                w_wait(k - 2, b % 4)
                g_start(k + 2, b % 4)

        for k, b in ((R - 2, 2), (R - 1, 3)):
            g_wait(k, b)
            w_start(k, b)
            w_wait(k - 2, (b + 2) % 4)
        w_wait(R - 2, 2)
        w_wait(R - 1, 3)

    return k(x2, W, pe)


def kernel(x, start_token, end_token, W):
    del start_token, end_token
    x2 = x.reshape(TOK // 128, 128)
    out = _embed(x2, W, _pos_encoding())
    return out.reshape(B, S, D)
